# baseline (speedup 1.0000x reference)
import functools

import jax
import jax.numpy as jnp
from jax import lax
from jax.experimental import pallas as pl
from jax.experimental.pallas import tpu as pltpu
from jax.experimental.pallas import tpu_sc as plsc

B = 4096
DPSI_THRESHOLD = 0.05
MSE_WEIGHT = 10.0
EPS = 1e-7
SQRT2 = 1.4142135623730951
LN2 = 0.6931471805599453

LANES = 16
NWORKERS = 32
ROWS_PER = B // NWORKERS
GROUPS = ROWS_PER // LANES
NCHUNK = B // LANES


def _ln(x):
    bits = lax.bitcast_convert_type(x, jnp.int32)
    e = ((bits >> 23) & 0xFF) - 127
    m = lax.bitcast_convert_type((bits & 0x7FFFFF) | 0x3F800000, jnp.float32)
    big = m > SQRT2
    m = jnp.where(big, m * 0.5, m)
    e = jnp.where(big, e + 1, e)
    z = (m - 1.0) / (m + 1.0)
    z2 = z * z
    p = z * (2.0 + z2 * (2.0 / 3.0 + z2 * (2.0 / 5.0 + z2 * (2.0 / 7.0))))
    return e.astype(jnp.float32) * LN2 + p


def _loss_body(pred_hbm, psi_hbm, ev_hbm, out,
               pred_v, psi_v, ev_v, u_band, bnd_v, stage_v):
    wid = lax.axis_index("s") * 2 + lax.axis_index("c")
    pltpu.sync_copy(pred_hbm, pred_v)
    pltpu.sync_copy(psi_hbm, psi_v)
    pltpu.sync_copy(ev_hbm, ev_v)

    base = wid * ROWS_PER
    cbase = wid * GROUPS
    lane = lax.iota(jnp.int32, LANES)
    zero = jnp.zeros((LANES,), jnp.float32)

    idx_hi = jnp.minimum(base + lane * LANES + (LANES - 1), B - 1)
    ev_ghi = plsc.load_gather(ev_v, [idx_hi])
    c_hi = jnp.zeros((LANES,), jnp.int32)
    for k in (256, 128, 64, 32, 16, 8, 4, 2, 1):
        nhi = c_hi + k
        cmin = plsc.load_gather(ev_v, [(jnp.minimum(nhi, NCHUNK) - 1) * LANES])
        c_hi = jnp.where((nhi <= NCHUNK) & (cmin <= ev_ghi), nhi, c_hi)
    bnd_v[...] = c_hi
    n_band = c_hi[GROUPS - 1] - cbase

    def band_body(j, b_bce):
        cb = (cbase + j) * LANES
        x = pred_v[pl.ds(cb, LANES)]
        y = psi_v[pl.ds(cb, LANES)]
        p = jnp.clip(y, EPS, 1.0 - EPS)
        u_band[pl.ds(j * LANES, LANES)] = x - _ln(p / (1.0 - p))
        terms = jnp.maximum(x, 0.0) - x * y + _ln(1.0 + jnp.exp(-jnp.abs(x)))
        keep = jnp.full((LANES,), j, jnp.int32) < GROUPS
        return b_bce + jnp.where(keep, terms, 0.0)

    acc_bce = lax.fori_loop(0, n_band, band_body, zero)

    def group_body(g, carry):
        rbase = base + g * LANES
        y = psi_v[pl.ds(rbase, LANES)]
        ev_r = ev_v[pl.ds(rbase, LANES)]
        u_r = u_band[pl.ds(g * LANES, LANES)]
        g_abs = cbase + g
        c_hi_g = bnd_v[pl.ds(g, LANES)][0]

        def chunk_body(c, acc):
            b_sq0, b_sq1, b_ct0, b_ct1 = acc
            wv = jnp.where(jnp.full((LANES,), c, jnp.int32) == g_abs, 1.0, 2.0)
            cb = c * LANES
            psi_c = psi_v[pl.ds(cb, LANES)]
            ev_c = ev_v[pl.ds(cb, LANES)]
            u_c = u_band[pl.ds((c - cbase) * LANES, LANES)]
            for s in range(LANES):
                idx = (lane + s) & (LANES - 1)
                u_x = u_c.at[idx].get(mode="promise_in_bounds")
                psi_x = psi_c.at[idx].get(mode="promise_in_bounds")
                ev_x = ev_c.at[idx].get(mode="promise_in_bounds")
                m = (ev_x == ev_r) & (jnp.abs(psi_x - y) >= DPSI_THRESHOLD)
                d = u_x - u_r
                sq = jnp.where(m, wv * (d * d), 0.0)
                ct = jnp.where(m, wv, 0.0)
                if s % 2 == 0:
                    b_sq0 = b_sq0 + sq
                    b_ct0 = b_ct0 + ct
                else:
                    b_sq1 = b_sq1 + sq
                    b_ct1 = b_ct1 + ct
            return (b_sq0, b_sq1, b_ct0, b_ct1)

        return lax.fori_loop(g_abs, c_hi_g, chunk_body, carry)

    acc = lax.fori_loop(0, GROUPS, group_body, (zero, zero, zero, zero))

    stage_v[pl.ds(0, LANES)] = acc_bce
    stage_v[pl.ds(LANES, LANES)] = acc[0] + acc[1]
    stage_v[pl.ds(2 * LANES, LANES)] = acc[2] + acc[3]
    pltpu.sync_copy(stage_v, out.at[wid])


_loss = functools.partial(
    pl.kernel,
    mesh=plsc.VectorSubcoreMesh(core_axis_name="c", subcore_axis_name="s"),
    compiler_params=pltpu.CompilerParams(needs_layout_passes=False),
    out_type=jax.ShapeDtypeStruct((NWORKERS, 3 * LANES), jnp.float32),
    scratch_types=[
        pltpu.VMEM((B,), jnp.float32),
        pltpu.VMEM((B,), jnp.float32),
        pltpu.VMEM((B,), jnp.int32),
        pltpu.VMEM((B,), jnp.float32),
        pltpu.VMEM((LANES,), jnp.int32),
        pltpu.VMEM((3 * LANES,), jnp.float32),
    ],
)(_loss_body)


def kernel(pred_psi_val, psi_val, event_id, use_BCE_loss_only):
    parts = _loss(pred_psi_val, psi_val, event_id.astype(jnp.int32))
    r = jnp.sum(parts.reshape(NWORKERS, 3, LANES), axis=(0, 2))
    bce = r[0] / B
    cnt = r[2]
    pairwise_mse = r[1] / jnp.maximum(cnt, 1.0)
    full_loss = bce + jnp.where(cnt > 0, pairwise_mse * MSE_WEIGHT, 0.0)
    return jnp.where(use_BCE_loss_only != 0, bce, full_loss)


# speedup vs baseline: 1.0075x; 1.0075x over previous
import functools

import jax
import jax.numpy as jnp
from jax import lax
from jax.experimental import pallas as pl
from jax.experimental.pallas import tpu as pltpu
from jax.experimental.pallas import tpu_sc as plsc

B = 4096
DPSI_THRESHOLD = 0.05
MSE_WEIGHT = 10.0
EPS = 1e-7
SQRT2 = 1.4142135623730951
LN2 = 0.6931471805599453

LANES = 16
NWORKERS = 32
ROWS_PER = B // NWORKERS
GROUPS = ROWS_PER // LANES
NCHUNK = B // LANES


def _ln(x):
    bits = lax.bitcast_convert_type(x, jnp.int32)
    e = ((bits >> 23) & 0xFF) - 127
    m = lax.bitcast_convert_type((bits & 0x7FFFFF) | 0x3F800000, jnp.float32)
    big = m > SQRT2
    m = jnp.where(big, m * 0.5, m)
    e = jnp.where(big, e + 1, e)
    z = (m - 1.0) / (m + 1.0)
    z2 = z * z
    p = z * (2.0 + z2 * (2.0 / 3.0 + z2 * (2.0 / 5.0 + z2 * (2.0 / 7.0))))
    return e.astype(jnp.float32) * LN2 + p


def _loss_body(pred_hbm, psi_hbm, ev_hbm, out,
               pred_v, psi_v, ev_v, u_band, bnd_v, stage_v, dma_sem):
    wid = lax.axis_index("s") * 2 + lax.axis_index("c")
    c1 = pltpu.async_copy(pred_hbm, pred_v, dma_sem)
    c2 = pltpu.async_copy(psi_hbm, psi_v, dma_sem)
    c3 = pltpu.async_copy(ev_hbm, ev_v, dma_sem)
    c1.wait()
    c2.wait()
    c3.wait()

    base = wid * ROWS_PER
    cbase = wid * GROUPS
    lane = lax.iota(jnp.int32, LANES)
    zero = jnp.zeros((LANES,), jnp.float32)

    idx_hi = jnp.minimum(base + lane * LANES + (LANES - 1), B - 1)
    ev_ghi = plsc.load_gather(ev_v, [idx_hi])
    c_hi = jnp.zeros((LANES,), jnp.int32)
    for k in (256, 128, 64, 32, 16, 8, 4, 2, 1):
        nhi = c_hi + k
        cmin = plsc.load_gather(ev_v, [(jnp.minimum(nhi, NCHUNK) - 1) * LANES])
        c_hi = jnp.where((nhi <= NCHUNK) & (cmin <= ev_ghi), nhi, c_hi)
    bnd_v[...] = c_hi
    n_band = c_hi[GROUPS - 1] - cbase

    def band_body(j, b_bce):
        cb = (cbase + j) * LANES
        x = pred_v[pl.ds(cb, LANES)]
        y = psi_v[pl.ds(cb, LANES)]
        p = jnp.clip(y, EPS, 1.0 - EPS)
        u_band[pl.ds(j * LANES, LANES)] = x - _ln(p / (1.0 - p))
        terms = jnp.maximum(x, 0.0) - x * y + _ln(1.0 + jnp.exp(-jnp.abs(x)))
        keep = jnp.full((LANES,), j, jnp.int32) < GROUPS
        return b_bce + jnp.where(keep, terms, 0.0)

    acc_bce = lax.fori_loop(0, n_band, band_body, zero)

    def group_body(g, carry):
        rbase = base + g * LANES
        y = psi_v[pl.ds(rbase, LANES)]
        ev_r = ev_v[pl.ds(rbase, LANES)]
        u_r = u_band[pl.ds(g * LANES, LANES)]
        g_abs = cbase + g
        c_hi_g = bnd_v[pl.ds(g, LANES)][0]

        def chunk_body(c, acc):
            b_sq0, b_sq1, b_ct0, b_ct1 = acc
            wv = jnp.where(jnp.full((LANES,), c, jnp.int32) == g_abs, 1.0, 2.0)
            cb = c * LANES
            psi_c = psi_v[pl.ds(cb, LANES)]
            ev_c = ev_v[pl.ds(cb, LANES)]
            u_c = u_band[pl.ds((c - cbase) * LANES, LANES)]
            for s in range(LANES):
                idx = (lane + s) & (LANES - 1)
                u_x = u_c.at[idx].get(mode="promise_in_bounds")
                psi_x = psi_c.at[idx].get(mode="promise_in_bounds")
                ev_x = ev_c.at[idx].get(mode="promise_in_bounds")
                m = (ev_x == ev_r) & (jnp.abs(psi_x - y) >= DPSI_THRESHOLD)
                d = u_x - u_r
                sq = jnp.where(m, wv * (d * d), 0.0)
                ct = jnp.where(m, wv, 0.0)
                if s % 2 == 0:
                    b_sq0 = b_sq0 + sq
                    b_ct0 = b_ct0 + ct
                else:
                    b_sq1 = b_sq1 + sq
                    b_ct1 = b_ct1 + ct
            return (b_sq0, b_sq1, b_ct0, b_ct1)

        return lax.fori_loop(g_abs, c_hi_g, chunk_body, carry)

    acc = lax.fori_loop(0, GROUPS, group_body, (zero, zero, zero, zero))

    stage_v[pl.ds(0, LANES)] = acc_bce
    stage_v[pl.ds(LANES, LANES)] = acc[0] + acc[1]
    stage_v[pl.ds(2 * LANES, LANES)] = acc[2] + acc[3]
    pltpu.sync_copy(stage_v, out.at[wid])


_loss = functools.partial(
    pl.kernel,
    mesh=plsc.VectorSubcoreMesh(core_axis_name="c", subcore_axis_name="s"),
    compiler_params=pltpu.CompilerParams(needs_layout_passes=False),
    out_type=jax.ShapeDtypeStruct((NWORKERS, 3 * LANES), jnp.float32),
    scratch_types=[
        pltpu.VMEM((B,), jnp.float32),
        pltpu.VMEM((B,), jnp.float32),
        pltpu.VMEM((B,), jnp.int32),
        pltpu.VMEM((B,), jnp.float32),
        pltpu.VMEM((LANES,), jnp.int32),
        pltpu.VMEM((3 * LANES,), jnp.float32),
        pltpu.SemaphoreType.DMA,
    ],
)(_loss_body)


def kernel(pred_psi_val, psi_val, event_id, use_BCE_loss_only):
    parts = _loss(pred_psi_val, psi_val, event_id.astype(jnp.int32))
    r = jnp.sum(parts.reshape(NWORKERS, 3, LANES), axis=(0, 2))
    bce = r[0] / B
    cnt = r[2]
    pairwise_mse = r[1] / jnp.maximum(cnt, 1.0)
    full_loss = bce + jnp.where(cnt > 0, pairwise_mse * MSE_WEIGHT, 0.0)
    return jnp.where(use_BCE_loss_only != 0, bce, full_loss)


# speedup vs baseline: 1.0460x; 1.0382x over previous
import functools

import jax
import jax.numpy as jnp
from jax import lax
from jax.experimental import pallas as pl
from jax.experimental.pallas import tpu as pltpu
from jax.experimental.pallas import tpu_sc as plsc

B = 4096
DPSI_THRESHOLD = 0.05
MSE_WEIGHT = 10.0
EPS = 1e-7
SQRT2 = 1.4142135623730951
LN2 = 0.6931471805599453

LANES = 16
NWORKERS = 32
ROWS_PER = B // NWORKERS
GROUPS = ROWS_PER // LANES
NCHUNK = B // LANES
BLK = 256
BLK_CH = BLK // LANES


def _ln(x):
    bits = lax.bitcast_convert_type(x, jnp.int32)
    e = ((bits >> 23) & 0xFF) - 127
    m = lax.bitcast_convert_type((bits & 0x7FFFFF) | 0x3F800000, jnp.float32)
    big = m > SQRT2
    m = jnp.where(big, m * 0.5, m)
    e = jnp.where(big, e + 1, e)
    z = (m - 1.0) / (m + 1.0)
    z2 = z * z
    p = z * (2.0 + z2 * (2.0 / 3.0 + z2 * (2.0 / 5.0 + z2 * (2.0 / 7.0))))
    return e.astype(jnp.float32) * LN2 + p


def _loss_body(pred_hbm, psi_hbm, ev_hbm, cf_hbm, out,
               pred_b, psi_b, ev_b, u_band, cf_v, bnd_v, stage_v, dma_sem):
    wid = lax.axis_index("s") * 2 + lax.axis_index("c")
    base = wid * ROWS_PER
    cbase = wid * GROUPS
    lane = lax.iota(jnp.int32, LANES)
    zero = jnp.zeros((LANES,), jnp.float32)

    c0 = pltpu.async_copy(cf_hbm, cf_v, dma_sem)
    c1 = pltpu.async_copy(pred_hbm.at[pl.ds(base, BLK)], pred_b.at[pl.ds(0, BLK)], dma_sem)
    c2 = pltpu.async_copy(psi_hbm.at[pl.ds(base, BLK)], psi_b.at[pl.ds(0, BLK)], dma_sem)
    c3 = pltpu.async_copy(ev_hbm.at[pl.ds(base, BLK)], ev_b.at[pl.ds(0, BLK)], dma_sem)
    c0.wait()
    c1.wait()
    c2.wait()
    c3.wait()

    ev_ghi = plsc.load_gather(ev_b, [lane * LANES + (LANES - 1)])
    c_hi = jnp.zeros((LANES,), jnp.int32)
    for k in (256, 128, 64, 32, 16, 8, 4, 2, 1):
        nhi = c_hi + k
        cmin = plsc.load_gather(cf_v, [jnp.minimum(nhi, NCHUNK) - 1])
        c_hi = jnp.where((nhi <= NCHUNK) & (cmin <= ev_ghi), nhi, c_hi)
    bnd_v[pl.ds(0, LANES)] = c_hi
    bnd_v[pl.ds(LANES, LANES)] = c_hi
    n_band = c_hi[GROUPS - 1] - cbase

    def blk_body(k, _):
        s = base + k * BLK
        d = k * BLK
        pltpu.sync_copy(pred_hbm.at[pl.ds(s, BLK)], pred_b.at[pl.ds(d, BLK)])
        pltpu.sync_copy(psi_hbm.at[pl.ds(s, BLK)], psi_b.at[pl.ds(d, BLK)])
        pltpu.sync_copy(ev_hbm.at[pl.ds(s, BLK)], ev_b.at[pl.ds(d, BLK)])
        return 0

    n_blk = (n_band + (BLK_CH - 1)) // BLK_CH
    lax.fori_loop(1, n_blk, blk_body, 0)

    def band_body(j, b_bce):
        cb = j * LANES
        x = pred_b[pl.ds(cb, LANES)]
        y = psi_b[pl.ds(cb, LANES)]
        p = jnp.clip(y, EPS, 1.0 - EPS)
        u_band[pl.ds(cb, LANES)] = x - _ln(p / (1.0 - p))
        terms = jnp.maximum(x, 0.0) - x * y + _ln(1.0 + jnp.exp(-jnp.abs(x)))
        keep = jnp.full((LANES,), j, jnp.int32) < GROUPS
        return b_bce + jnp.where(keep, terms, 0.0)

    acc_bce = lax.fori_loop(0, n_band, band_body, zero)

    def group_body(g, carry):
        gb = g * LANES
        y = psi_b[pl.ds(gb, LANES)]
        ev_r = ev_b[pl.ds(gb, LANES)]
        u_r = u_band[pl.ds(gb, LANES)]
        g_abs = cbase + g
        c_hi_g = bnd_v[pl.ds(g, LANES)][0]

        def chunk_body(c, acc):
            b_sq0, b_sq1, b_ct0, b_ct1 = acc
            wv = jnp.where(jnp.full((LANES,), c, jnp.int32) == g_abs, 1.0, 2.0)
            cb = (c - cbase) * LANES
            psi_c = psi_b[pl.ds(cb, LANES)]
            ev_c = ev_b[pl.ds(cb, LANES)]
            u_c = u_band[pl.ds(cb, LANES)]
            for s in range(LANES):
                idx = (lane + s) & (LANES - 1)
                u_x = u_c.at[idx].get(mode="promise_in_bounds")
                psi_x = psi_c.at[idx].get(mode="promise_in_bounds")
                ev_x = ev_c.at[idx].get(mode="promise_in_bounds")
                m = (ev_x == ev_r) & (jnp.abs(psi_x - y) >= DPSI_THRESHOLD)
                d = u_x - u_r
                sq = jnp.where(m, wv * (d * d), 0.0)
                ct = jnp.where(m, wv, 0.0)
                if s % 2 == 0:
                    b_sq0 = b_sq0 + sq
                    b_ct0 = b_ct0 + ct
                else:
                    b_sq1 = b_sq1 + sq
                    b_ct1 = b_ct1 + ct
            return (b_sq0, b_sq1, b_ct0, b_ct1)

        return lax.fori_loop(g_abs, c_hi_g, chunk_body, carry)

    acc = lax.fori_loop(0, GROUPS, group_body, (zero, zero, zero, zero))

    stage_v[pl.ds(0, LANES)] = acc_bce
    stage_v[pl.ds(LANES, LANES)] = acc[0] + acc[1]
    stage_v[pl.ds(2 * LANES, LANES)] = acc[2] + acc[3]
    pltpu.sync_copy(stage_v, out.at[wid])


_loss = functools.partial(
    pl.kernel,
    mesh=plsc.VectorSubcoreMesh(core_axis_name="c", subcore_axis_name="s"),
    compiler_params=pltpu.CompilerParams(needs_layout_passes=False),
    out_type=jax.ShapeDtypeStruct((NWORKERS, 3 * LANES), jnp.float32),
    scratch_types=[
        pltpu.VMEM((B,), jnp.float32),
        pltpu.VMEM((B,), jnp.float32),
        pltpu.VMEM((B,), jnp.int32),
        pltpu.VMEM((B,), jnp.float32),
        pltpu.VMEM((NCHUNK,), jnp.int32),
        pltpu.VMEM((2 * LANES,), jnp.int32),
        pltpu.VMEM((3 * LANES,), jnp.float32),
        pltpu.SemaphoreType.DMA,
    ],
)(_loss_body)


def kernel(pred_psi_val, psi_val, event_id, use_BCE_loss_only):
    ev = event_id.astype(jnp.int32)
    cf = ev[::LANES]
    pred_p = jnp.pad(pred_psi_val, (0, BLK))
    psi_p = jnp.pad(psi_val, (0, BLK), constant_values=0.5)
    ev_p = jnp.pad(ev, (0, BLK), constant_values=-1)
    parts = _loss(pred_p, psi_p, ev_p, cf)
    r = jnp.sum(parts.reshape(NWORKERS, 3, LANES), axis=(0, 2))
    bce = r[0] / B
    cnt = r[2]
    pairwise_mse = r[1] / jnp.maximum(cnt, 1.0)
    full_loss = bce + jnp.where(cnt > 0, pairwise_mse * MSE_WEIGHT, 0.0)
    return jnp.where(use_BCE_loss_only != 0, bce, full_loss)


# speedup vs baseline: 1.0735x; 1.0263x over previous
import functools

import jax
import jax.numpy as jnp
from jax import lax
from jax.experimental import pallas as pl
from jax.experimental.pallas import tpu as pltpu
from jax.experimental.pallas import tpu_sc as plsc

B = 4096
DPSI_THRESHOLD = 0.05
MSE_WEIGHT = 10.0
EPS = 1e-7
SQRT2 = 1.4142135623730951
LN2 = 0.6931471805599453

LANES = 16
NWORKERS = 32
ROWS_PER = B // NWORKERS
GROUPS = ROWS_PER // LANES
NCHUNK = B // LANES
BLK = 128
BLK_CH = BLK // LANES


def _ln(x):
    bits = lax.bitcast_convert_type(x, jnp.int32)
    e = ((bits >> 23) & 0xFF) - 127
    m = lax.bitcast_convert_type((bits & 0x7FFFFF) | 0x3F800000, jnp.float32)
    big = m > SQRT2
    m = jnp.where(big, m * 0.5, m)
    e = jnp.where(big, e + 1, e)
    z = (m - 1.0) / (m + 1.0)
    z2 = z * z
    p = z * (2.0 + z2 * (2.0 / 3.0 + z2 * (2.0 / 5.0 + z2 * (2.0 / 7.0))))
    return e.astype(jnp.float32) * LN2 + p


def _loss_body(pred_hbm, psi_hbm, ev_hbm, cf_hbm, out,
               pred_b, psi_b, ev_b, u_band, cf_v, bnd_v, stage_v, dma_sem):
    wid = lax.axis_index("s") * 2 + lax.axis_index("c")
    base = wid * ROWS_PER
    cbase = wid * GROUPS
    lane = lax.iota(jnp.int32, LANES)
    zero = jnp.zeros((LANES,), jnp.float32)

    c0 = pltpu.async_copy(cf_hbm, cf_v, dma_sem)
    c1 = pltpu.async_copy(pred_hbm.at[pl.ds(base, BLK)], pred_b.at[pl.ds(0, BLK)], dma_sem)
    c2 = pltpu.async_copy(psi_hbm.at[pl.ds(base, BLK)], psi_b.at[pl.ds(0, BLK)], dma_sem)
    c3 = pltpu.async_copy(ev_hbm.at[pl.ds(base, BLK)], ev_b.at[pl.ds(0, BLK)], dma_sem)
    c0.wait()
    c1.wait()
    c2.wait()
    c3.wait()

    ev_ghi = plsc.load_gather(ev_b, [jnp.minimum(lane, GROUPS - 1) * LANES + (LANES - 1)])
    c_hi = jnp.zeros((LANES,), jnp.int32)
    for k in (256, 128, 64, 32, 16, 8, 4, 2, 1):
        nhi = c_hi + k
        cmin = plsc.load_gather(cf_v, [jnp.minimum(nhi, NCHUNK) - 1])
        c_hi = jnp.where((nhi <= NCHUNK) & (cmin <= ev_ghi), nhi, c_hi)
    bnd_v[pl.ds(0, LANES)] = c_hi
    bnd_v[pl.ds(LANES, LANES)] = c_hi
    n_band = c_hi[GROUPS - 1] - cbase

    def blk_body(k, _):
        s = jnp.minimum(base + k * BLK, B - BLK)
        d = s - base
        b1 = pltpu.async_copy(pred_hbm.at[pl.ds(s, BLK)], pred_b.at[pl.ds(d, BLK)], dma_sem)
        b2 = pltpu.async_copy(psi_hbm.at[pl.ds(s, BLK)], psi_b.at[pl.ds(d, BLK)], dma_sem)
        b3 = pltpu.async_copy(ev_hbm.at[pl.ds(s, BLK)], ev_b.at[pl.ds(d, BLK)], dma_sem)
        b1.wait()
        b2.wait()
        b3.wait()
        return 0

    n_blk = (n_band + (BLK_CH - 1)) // BLK_CH
    lax.fori_loop(1, n_blk, blk_body, 0)

    def band_body(j, b_bce):
        cb = j * LANES
        x = pred_b[pl.ds(cb, LANES)]
        y = psi_b[pl.ds(cb, LANES)]
        p = jnp.clip(y, EPS, 1.0 - EPS)
        u_band[pl.ds(cb, LANES)] = x - _ln(p / (1.0 - p))
        terms = jnp.maximum(x, 0.0) - x * y + _ln(1.0 + jnp.exp(-jnp.abs(x)))
        keep = jnp.full((LANES,), j, jnp.int32) < GROUPS
        return b_bce + jnp.where(keep, terms, 0.0)

    acc_bce = lax.fori_loop(0, n_band, band_body, zero)

    def group_body(g, carry):
        gb = g * LANES
        y = psi_b[pl.ds(gb, LANES)]
        ev_r = ev_b[pl.ds(gb, LANES)]
        u_r = u_band[pl.ds(gb, LANES)]
        g_abs = cbase + g
        c_hi_g = bnd_v[pl.ds(g, LANES)][0]

        def chunk_body(c, acc):
            b_sq0, b_sq1, b_ct0, b_ct1 = acc
            wv = jnp.where(jnp.full((LANES,), c, jnp.int32) == g_abs, 1.0, 2.0)
            cb = (c - cbase) * LANES
            psi_c = psi_b[pl.ds(cb, LANES)]
            ev_c = ev_b[pl.ds(cb, LANES)]
            u_c = u_band[pl.ds(cb, LANES)]
            for s in range(LANES):
                idx = (lane + s) & (LANES - 1)
                u_x = u_c.at[idx].get(mode="promise_in_bounds")
                psi_x = psi_c.at[idx].get(mode="promise_in_bounds")
                ev_x = ev_c.at[idx].get(mode="promise_in_bounds")
                m = (ev_x == ev_r) & (jnp.abs(psi_x - y) >= DPSI_THRESHOLD)
                d = u_x - u_r
                sq = jnp.where(m, wv * (d * d), 0.0)
                ct = jnp.where(m, wv, 0.0)
                if s % 2 == 0:
                    b_sq0 = b_sq0 + sq
                    b_ct0 = b_ct0 + ct
                else:
                    b_sq1 = b_sq1 + sq
                    b_ct1 = b_ct1 + ct
            return (b_sq0, b_sq1, b_ct0, b_ct1)

        return lax.fori_loop(g_abs, c_hi_g, chunk_body, carry)

    acc = lax.fori_loop(0, GROUPS, group_body, (zero, zero, zero, zero))

    stage_v[pl.ds(0, LANES)] = acc_bce
    stage_v[pl.ds(LANES, LANES)] = acc[0] + acc[1]
    stage_v[pl.ds(2 * LANES, LANES)] = acc[2] + acc[3]
    pltpu.sync_copy(stage_v, out.at[wid])


_loss = functools.partial(
    pl.kernel,
    mesh=plsc.VectorSubcoreMesh(core_axis_name="c", subcore_axis_name="s"),
    compiler_params=pltpu.CompilerParams(needs_layout_passes=False),
    out_type=jax.ShapeDtypeStruct((NWORKERS, 3 * LANES), jnp.float32),
    scratch_types=[
        pltpu.VMEM((B,), jnp.float32),
        pltpu.VMEM((B,), jnp.float32),
        pltpu.VMEM((B,), jnp.int32),
        pltpu.VMEM((B,), jnp.float32),
        pltpu.VMEM((NCHUNK,), jnp.int32),
        pltpu.VMEM((2 * LANES,), jnp.int32),
        pltpu.VMEM((3 * LANES,), jnp.float32),
        pltpu.SemaphoreType.DMA,
    ],
)(_loss_body)


def kernel(pred_psi_val, psi_val, event_id, use_BCE_loss_only):
    ev = event_id.astype(jnp.int32)
    cf = ev[::LANES]
    parts = _loss(pred_psi_val, psi_val, ev, cf)
    r = jnp.sum(parts.reshape(NWORKERS, 3, LANES), axis=(0, 2))
    bce = r[0] / B
    cnt = r[2]
    pairwise_mse = r[1] / jnp.maximum(cnt, 1.0)
    full_loss = bce + jnp.where(cnt > 0, pairwise_mse * MSE_WEIGHT, 0.0)
    return jnp.where(use_BCE_loss_only != 0, bce, full_loss)
